# initial kernel scaffold (unmeasured)
import jax
import jax.numpy as jnp
from jax import lax
from jax.experimental import pallas as pl
from jax.experimental.pallas import tpu as pltpu


def kernel(
    x,
):
    def body(*refs):
        pass

    out_shape = jax.ShapeDtypeStruct(..., jnp.float32)
    return pl.pallas_call(body, out_shape=out_shape)(...)


# baseline (device time: 12779 ns/iter reference)
import jax
import jax.numpy as jnp
from jax import lax
from jax.experimental import pallas as pl
from jax.experimental.pallas import tpu as pltpu

N_DEV = 32


def kernel(x):
    m_per, n = x.shape

    def body(x_ref, out_ref, gather_ref, send_sems, recv_sems):
        my = lax.axis_index("i")

        barrier_sem = pltpu.get_barrier_semaphore()
        for d in range(1, N_DEV):
            pl.semaphore_signal(
                barrier_sem,
                inc=1,
                device_id=(lax.rem(my + d, N_DEV),),
                device_id_type=pl.DeviceIdType.MESH,
            )
        pl.semaphore_wait(barrier_sem, N_DEV - 1)

        vals = x_ref[:, :]
        local_max = jnp.max(vals, axis=0)
        row_ids = lax.broadcasted_iota(jnp.int32, (m_per, n), 0)
        cand = jnp.where(vals == local_max[None, :], row_ids, jnp.int32(m_per))
        local_arg = jnp.min(cand, axis=0)
        global_idx = (my * m_per + local_arg).astype(jnp.float32)

        gather_ref[0, 0, :] = local_max
        gather_ref[0, 1, :] = global_idx

        rdmas = []
        for d in range(1, N_DEV):
            rdma = pltpu.make_async_remote_copy(
                src_ref=gather_ref.at[0],
                dst_ref=gather_ref.at[d],
                send_sem=send_sems.at[d],
                recv_sem=recv_sems.at[d],
                device_id=(lax.rem(my + d, N_DEV),),
                device_id_type=pl.DeviceIdType.MESH,
            )
            rdma.start()
            rdmas.append(rdma)

        for rdma in rdmas:
            rdma.wait_recv()

        all_vals = gather_ref[:, 0, :]
        all_idx = gather_ref[:, 1, :]
        gmax = jnp.max(all_vals, axis=0)
        cand2 = jnp.where(all_vals == gmax[None, :], all_idx, jnp.float32(1e9))
        gidx = jnp.min(cand2, axis=0)

        out_ref[0, :] = gmax
        out_ref[1, :] = gidx

        for rdma in rdmas:
            rdma.wait_send()

    return pl.pallas_call(
        body,
        out_shape=jax.ShapeDtypeStruct((2, n), jnp.float32),
        in_specs=[pl.BlockSpec(memory_space=pltpu.VMEM)],
        out_specs=pl.BlockSpec(memory_space=pltpu.VMEM),
        scratch_shapes=[
            pltpu.VMEM((N_DEV, 2, n), jnp.float32),
            pltpu.SemaphoreType.DMA((N_DEV,)),
            pltpu.SemaphoreType.DMA((N_DEV,)),
        ],
        compiler_params=pltpu.CompilerParams(collective_id=0),
    )(x)


# device time: 12631 ns/iter; 1.0117x vs baseline; 1.0117x over previous
import jax
import jax.numpy as jnp
from jax import lax
from jax.experimental import pallas as pl
from jax.experimental.pallas import tpu as pltpu

N_DEV = 32


def kernel(x):
    m_per, n = x.shape

    def body(x_ref, out_ref, gather_ref, send_sems, recv_sems):
        my = lax.axis_index("i")

        barrier_sem = pltpu.get_barrier_semaphore()
        for d in range(1, N_DEV):
            pl.semaphore_signal(
                barrier_sem,
                inc=1,
                device_id=(lax.rem(my + d, N_DEV),),
                device_id_type=pl.DeviceIdType.MESH,
            )

        vals = x_ref[:, :]
        local_max = jnp.max(vals, axis=0)
        row_ids = lax.broadcasted_iota(jnp.int32, (m_per, n), 0)
        cand = jnp.where(vals == local_max[None, :], row_ids, jnp.int32(m_per))
        local_arg = jnp.min(cand, axis=0)
        global_idx = (my * m_per + local_arg).astype(jnp.float32)

        gather_ref[0, 0, :] = local_max
        gather_ref[0, 1, :] = global_idx

        pl.semaphore_wait(barrier_sem, N_DEV - 1)

        rdmas = []
        for d in range(1, N_DEV):
            rdma = pltpu.make_async_remote_copy(
                src_ref=gather_ref.at[0],
                dst_ref=gather_ref.at[d],
                send_sem=send_sems.at[d],
                recv_sem=recv_sems.at[d],
                device_id=(lax.rem(my + d, N_DEV),),
                device_id_type=pl.DeviceIdType.MESH,
            )
            rdma.start()
            rdmas.append(rdma)

        for rdma in rdmas:
            rdma.wait_recv()

        all_vals = gather_ref[:, 0, :]
        all_idx = gather_ref[:, 1, :]
        gmax = jnp.max(all_vals, axis=0)
        cand2 = jnp.where(all_vals == gmax[None, :], all_idx, jnp.float32(1e9))
        gidx = jnp.min(cand2, axis=0)

        out_ref[0, :] = gmax
        out_ref[1, :] = gidx

        for rdma in rdmas:
            rdma.wait_send()

    return pl.pallas_call(
        body,
        out_shape=jax.ShapeDtypeStruct((2, n), jnp.float32),
        in_specs=[pl.BlockSpec(memory_space=pltpu.VMEM)],
        out_specs=pl.BlockSpec(memory_space=pltpu.VMEM),
        scratch_shapes=[
            pltpu.VMEM((N_DEV, 2, n), jnp.float32),
            pltpu.SemaphoreType.DMA((N_DEV,)),
            pltpu.SemaphoreType.DMA((N_DEV,)),
        ],
        compiler_params=pltpu.CompilerParams(collective_id=0),
    )(x)


# device time: 12589 ns/iter; 1.0151x vs baseline; 1.0033x over previous
import jax
import jax.numpy as jnp
from jax import lax
from jax.experimental import pallas as pl
from jax.experimental.pallas import tpu as pltpu

N_DEV = 32


def kernel(x):
    m_per, n = x.shape

    def body(x_ref, out_ref, gather_ref, send_sems, recv_sems):
        my = lax.axis_index("i")

        barrier_sem = pltpu.get_barrier_semaphore()
        for d in range(1, N_DEV):
            pl.semaphore_signal(
                barrier_sem,
                inc=1,
                device_id=(lax.rem(my + d, N_DEV),),
                device_id_type=pl.DeviceIdType.MESH,
            )

        vals = x_ref[:, :]
        local_max = jnp.max(vals, axis=0)
        row_ids = lax.broadcasted_iota(jnp.int32, (m_per, n), 0)
        cand = jnp.where(vals == local_max[None, :], row_ids, jnp.int32(m_per))
        local_arg = jnp.min(cand, axis=0)
        global_idx = (my * m_per + local_arg).astype(jnp.float32)

        gather_ref[0, 0, :] = local_max
        gather_ref[0, 1, :] = global_idx

        pl.semaphore_wait(barrier_sem, N_DEV - 1)

        rdmas = []
        for d in range(1, N_DEV):
            rdma = pltpu.make_async_remote_copy(
                src_ref=gather_ref.at[0],
                dst_ref=gather_ref.at[d],
                send_sem=send_sems.at[d],
                recv_sem=recv_sems.at[d],
                device_id=(lax.rem(my + d, N_DEV),),
                device_id_type=pl.DeviceIdType.MESH,
            )
            rdma.start()
            rdmas.append(rdma)

        acc_v = local_max
        acc_i = global_idx
        bounds = [1, 9, 17, 25, N_DEV]
        for g in range(len(bounds) - 1):
            lo, hi = bounds[g], bounds[g + 1]
            for d in range(lo, hi):
                rdmas[d - 1].wait_recv()
            vals_g = gather_ref[lo:hi, 0, :]
            idx_g = gather_ref[lo:hi, 1, :]
            gmax_g = jnp.max(vals_g, axis=0)
            cand_g = jnp.where(
                vals_g == gmax_g[None, :], idx_g, jnp.float32(1e9)
            )
            gidx_g = jnp.min(cand_g, axis=0)
            new_v = jnp.maximum(acc_v, gmax_g)
            acc_i = jnp.where(
                gmax_g > acc_v,
                gidx_g,
                jnp.where(gmax_g == acc_v, jnp.minimum(acc_i, gidx_g), acc_i),
            )
            acc_v = new_v

        out_ref[0, :] = acc_v
        out_ref[1, :] = acc_i

        for rdma in rdmas:
            rdma.wait_send()

    return pl.pallas_call(
        body,
        out_shape=jax.ShapeDtypeStruct((2, n), jnp.float32),
        in_specs=[pl.BlockSpec(memory_space=pltpu.VMEM)],
        out_specs=pl.BlockSpec(memory_space=pltpu.VMEM),
        scratch_shapes=[
            pltpu.VMEM((N_DEV, 2, n), jnp.float32),
            pltpu.SemaphoreType.DMA((N_DEV,)),
            pltpu.SemaphoreType.DMA((N_DEV,)),
        ],
        compiler_params=pltpu.CompilerParams(collective_id=0),
    )(x)
